# Initial kernel scaffold; baseline (speedup 1.0000x reference)
import jax
import jax.numpy as jnp
from jax.experimental import pallas as pl


def kernel(Q, K, V, W_w, U_w, V_w, batch_index):
    raise NotImplementedError("write your pallas kernel here")


# speedup vs baseline: 4.9020x; 4.9020x over previous
import functools

import jax
import jax.numpy as jnp
from jax import lax
from jax.experimental import pallas as pl
from jax.experimental.pallas import tpu as pltpu
from jax.experimental.pallas import tpu_sc as plsc

N = 320000
NUM_SEG = 10000
Q_SIZE = 128
K_SIZE = 128
HID = 64
D_V = 128


BLK = 1280
GRID1 = N // BLK
EROWS = BLK // 128


def _tc_dense_body(q_ref, k_ref, v_ref, w_ref, u_ref, vw_ref, p_ref, e_ref):
    qw = jnp.dot(q_ref[...], w_ref[...], preferred_element_type=jnp.float32)
    ku = jnp.dot(k_ref[...], u_ref[...], preferred_element_type=jnp.float32)
    t = jnp.tanh(qw + ku)
    beta = jnp.sum(t * vw_ref[0:1, :], axis=1, keepdims=True)
    e = jnp.exp(beta)
    p_ref[...] = v_ref[...] * e
    e_ref[...] = jnp.reshape(e[:, 0], (1, EROWS, 128))


def _tc_dense(Q, K, V, W_w, U_w, vw8):
    return pl.pallas_call(
        _tc_dense_body,
        grid=(GRID1,),
        in_specs=[
            pl.BlockSpec((BLK, Q_SIZE), lambda i: (i, 0)),
            pl.BlockSpec((BLK, K_SIZE), lambda i: (i, 0)),
            pl.BlockSpec((BLK, D_V), lambda i: (i, 0)),
            pl.BlockSpec((Q_SIZE, HID), lambda i: (0, 0)),
            pl.BlockSpec((K_SIZE, HID), lambda i: (0, 0)),
            pl.BlockSpec((8, HID), lambda i: (0, 0)),
        ],
        out_specs=[
            pl.BlockSpec((BLK, D_V), lambda i: (i, 0)),
            pl.BlockSpec((1, EROWS, 128), lambda i: (i, 0, 0)),
        ],
        out_shape=[
            jax.ShapeDtypeStruct((N, D_V), jnp.float32),
            jax.ShapeDtypeStruct((GRID1, EROWS, 128), jnp.float32),
        ],
    )(Q, K, V, W_w, U_w, vw8)


NCORE = 2
NSUB = 16
NW = NCORE * NSUB
CH = 128
NCHUNK = N // CH
CHUNK_ITERS = -(-NCHUNK // NW)
SEG_PER_TILE = 624
SEG_TAIL = NUM_SEG - NSUB * SEG_PER_TILE
ZR = 104


def _sc_scatter_body(p_hbm, e_hbm, idx_hbm, zeros_hbm,
                     acc_out, den_out,
                     data_v, e_v, idx_v, den_v, acc_sh):
    cid = lax.axis_index("c")
    sid = lax.axis_index("s")
    wid = cid * NSUB + sid

    for z in range(SEG_PER_TILE // ZR):
        pltpu.sync_copy(zeros_hbm,
                        acc_sh.at[pl.ds(sid * SEG_PER_TILE + z * ZR, ZR)])

    @pl.when(sid == NSUB - 1)
    def _():
        pltpu.sync_copy(zeros_hbm.at[pl.ds(0, SEG_TAIL)],
                        acc_sh.at[pl.ds(NSUB * SEG_PER_TILE, SEG_TAIL)])

    def _zero_den(i, carry):
        den_v[pl.ds(i * 16, 16)] = jnp.zeros((16,), jnp.float32)
        return carry
    lax.fori_loop(0, NUM_SEG // 16, _zero_den, 0)

    plsc.subcore_barrier()

    def _chunk(i, carry):
        g = i * NW + wid

        @pl.when(g < NCHUNK)
        def _():
            base = g * CH
            pltpu.sync_copy(p_hbm.at[pl.ds(base, CH)], data_v)
            pltpu.sync_copy(e_hbm.at[pl.ds(g, 1)], e_v)
            pltpu.sync_copy(idx_hbm.at[pl.ds(base, CH)], idx_v)
            pltpu.sync_copy(data_v, acc_sh.at[idx_v], add=True)
            for j in range(CH // 16):
                sl = pl.ds(j * 16, 16)
                plsc.addupdate_scatter(den_v, [idx_v[sl]], e_v[0, sl])
        return carry

    lax.fori_loop(0, CHUNK_ITERS, _chunk, 0)

    plsc.subcore_barrier()

    row0 = sid * SEG_PER_TILE
    pltpu.sync_copy(acc_sh.at[pl.ds(row0, SEG_PER_TILE)],
                    acc_out.at[cid, pl.ds(row0, SEG_PER_TILE)])

    @pl.when(sid == NSUB - 1)
    def _():
        pltpu.sync_copy(acc_sh.at[pl.ds(NSUB * SEG_PER_TILE, SEG_TAIL)],
                        acc_out.at[cid, pl.ds(NSUB * SEG_PER_TILE, SEG_TAIL)])

    pltpu.sync_copy(den_v, den_out.at[pl.ds(wid * NUM_SEG, NUM_SEG)])


def _sc_scatter(P, E2, idx, zeros):
    f = functools.partial(
        pl.kernel,
        mesh=plsc.VectorSubcoreMesh(core_axis_name="c", subcore_axis_name="s"),
        compiler_params=pltpu.CompilerParams(needs_layout_passes=False),
        out_type=[
            jax.ShapeDtypeStruct((NCORE, NUM_SEG, D_V), jnp.float32),
            jax.ShapeDtypeStruct((NW * NUM_SEG,), jnp.float32),
        ],
        scratch_types=[
            pltpu.VMEM((CH, D_V), jnp.float32),
            pltpu.VMEM((1, CH), jnp.float32),
            pltpu.VMEM((CH,), jnp.int32),
            pltpu.VMEM((NUM_SEG,), jnp.float32),
            pltpu.VMEM_SHARED((NUM_SEG, D_V), jnp.float32),
        ],
    )(_sc_scatter_body)
    return f(P, E2, idx, zeros)


def _tc_combine_body(acc_ref, den_ref, out_ref):
    num = acc_ref[0] + acc_ref[1]
    den = jnp.sum(den_ref[...], axis=0)[:, None]
    out_ref[...] = jnp.where(den > 0.0, num / den, 0.0)


def _tc_combine(acc, den):
    return pl.pallas_call(
        _tc_combine_body,
        out_shape=jax.ShapeDtypeStruct((NUM_SEG, D_V), jnp.float32),
    )(acc, den)


def kernel(Q, K, V, W_w, U_w, V_w, batch_index):
    vw8 = jnp.broadcast_to(V_w.reshape(1, HID), (8, HID))
    idx = batch_index.astype(jnp.int32)
    zeros = jnp.zeros((ZR, D_V), jnp.float32)
    P, E2 = _tc_dense(Q, K, V, W_w, U_w, vw8)
    acc, den = _sc_scatter(P, E2.reshape(N // 128, 128), idx, zeros)
    return _tc_combine(acc, den.reshape(NW, NUM_SEG))


# speedup vs baseline: 6.4451x; 1.3148x over previous
import functools

import jax
import jax.numpy as jnp
from jax import lax
from jax.experimental import pallas as pl
from jax.experimental.pallas import tpu as pltpu
from jax.experimental.pallas import tpu_sc as plsc

N = 320000
NUM_SEG = 10000
Q_SIZE = 128
K_SIZE = 128
HID = 64
D_V = 128


BLK = 1280
GRID1 = N // BLK
EROWS = BLK // 128


def _tc_dense_body(q_ref, k_ref, v_ref, w_ref, u_ref, vw_ref, p_ref, e_ref):
    qw = jnp.dot(q_ref[...], w_ref[...], preferred_element_type=jnp.float32)
    ku = jnp.dot(k_ref[...], u_ref[...], preferred_element_type=jnp.float32)
    t = jnp.tanh(qw + ku)
    beta = jnp.sum(t * vw_ref[0:1, :], axis=1, keepdims=True)
    e = jnp.exp(beta)
    p_ref[...] = v_ref[...] * e
    e_ref[...] = jnp.reshape(e[:, 0], (1, EROWS, 128))


def _tc_dense(Q, K, V, W_w, U_w, vw8):
    return pl.pallas_call(
        _tc_dense_body,
        grid=(GRID1,),
        in_specs=[
            pl.BlockSpec((BLK, Q_SIZE), lambda i: (i, 0)),
            pl.BlockSpec((BLK, K_SIZE), lambda i: (i, 0)),
            pl.BlockSpec((BLK, D_V), lambda i: (i, 0)),
            pl.BlockSpec((Q_SIZE, HID), lambda i: (0, 0)),
            pl.BlockSpec((K_SIZE, HID), lambda i: (0, 0)),
            pl.BlockSpec((8, HID), lambda i: (0, 0)),
        ],
        out_specs=[
            pl.BlockSpec((BLK, D_V), lambda i: (i, 0)),
            pl.BlockSpec((1, EROWS, 128), lambda i: (i, 0, 0)),
        ],
        out_shape=[
            jax.ShapeDtypeStruct((N, D_V), jnp.float32),
            jax.ShapeDtypeStruct((GRID1, EROWS, 128), jnp.float32),
        ],
    )(Q, K, V, W_w, U_w, vw8)


NCORE = 2
NSUB = 16
NW = NCORE * NSUB
CH = 80
ROWS_PER_TILE = N // NW
ITERS = ROWS_PER_TILE // CH
NBUF = 3
PREF = 2
DEN_ROWS = 80
SEG_PER_TILE = 624
SEG_TAIL = NUM_SEG - NSUB * SEG_PER_TILE
ZR = 104


def _sc_scatter_body(p_hbm, e_hbm, idx_hbm, zeros_hbm,
                     acc_out, den_out, *rest):
    data = list(rest[0:NBUF])
    ev = list(rest[NBUF:2 * NBUF])
    xv = list(rest[2 * NBUF:3 * NBUF])
    den_v = rest[3 * NBUF]
    idx_id = rest[3 * NBUF + 1]
    acc_sh = rest[3 * NBUF + 2]
    den_sh = rest[3 * NBUF + 3]
    isem = list(rest[3 * NBUF + 4:4 * NBUF + 4])
    ssem = list(rest[4 * NBUF + 4:5 * NBUF + 4])

    cid = lax.axis_index("c")
    sid = lax.axis_index("s")
    wid = cid * NSUB + sid
    row0 = wid * ROWS_PER_TILE

    def in_copies(i, b):
        base = row0 + i * CH
        return (
            pltpu.make_async_copy(p_hbm.at[pl.ds(base, CH)], data[b], isem[b]),
            pltpu.make_async_copy(e_hbm.at[pl.ds(base, CH)], ev[b], isem[b]),
            pltpu.make_async_copy(idx_hbm.at[pl.ds(base, CH)], xv[b], isem[b]),
        )

    def scat_copy(b):
        return pltpu.make_async_copy(data[b], acc_sh.at[xv[b]], ssem[b])

    for z in range(SEG_PER_TILE // ZR):
        pltpu.sync_copy(zeros_hbm,
                        acc_sh.at[pl.ds(sid * SEG_PER_TILE + z * ZR, ZR)])

    @pl.when(sid == NSUB - 1)
    def _():
        pltpu.sync_copy(zeros_hbm.at[pl.ds(0, SEG_TAIL)],
                        acc_sh.at[pl.ds(NSUB * SEG_PER_TILE, SEG_TAIL)])

    pltpu.sync_copy(zeros_hbm.at[pl.ds(0, DEN_ROWS)], den_v)
    for j in range(DEN_ROWS // 16):
        idx_id[pl.ds(j * 16, 16)] = lax.iota(jnp.int32, 16) + (j * 16)

    @pl.when(sid == 0)
    def _():
        pltpu.sync_copy(zeros_hbm.at[pl.ds(0, DEN_ROWS)], den_sh)

    plsc.subcore_barrier()

    for b in range(PREF):
        for c in in_copies(b, b):
            c.start()

    def _step(i, b):
        for c in in_copies(i, b):
            c.wait()
        for j in range(CH // 16):
            sl = pl.ds(j * 16, 16)
            idx16 = xv[b][sl]
            hi16 = lax.shift_right_logical(idx16, 7)
            lo16 = lax.bitwise_and(idx16, 127)
            plsc.addupdate_scatter(den_v, [hi16, lo16], ev[b][sl])
        scat_copy(b).start(add=True)

    LAG = NBUF - PREF

    def _round(o, carry):
        for b in range(NBUF):
            i = o * NBUF + b
            _step(i, b)
            bp = (b + PREF) % NBUF

            @pl.when(i >= LAG)
            def _():
                scat_copy(bp).wait()

            @pl.when(i + PREF < ITERS)
            def _():
                for c in in_copies(i + PREF, bp):
                    c.start()
        return carry

    MAIN = (ITERS // NBUF) * NBUF
    lax.fori_loop(0, ITERS // NBUF, _round, 0)

    for i in range(MAIN, ITERS):
        _step(i, i % NBUF)
        scat_copy((i + PREF) % NBUF).wait()

    for i in range(ITERS - LAG, ITERS):
        scat_copy(i % NBUF).wait()

    pltpu.sync_copy(den_v, den_sh.at[idx_id], add=True)

    plsc.subcore_barrier()

    row0 = sid * SEG_PER_TILE
    pltpu.sync_copy(acc_sh.at[pl.ds(row0, SEG_PER_TILE)],
                    acc_out.at[cid, pl.ds(row0, SEG_PER_TILE)])

    @pl.when(sid == NSUB - 1)
    def _():
        pltpu.sync_copy(acc_sh.at[pl.ds(NSUB * SEG_PER_TILE, SEG_TAIL)],
                        acc_out.at[cid, pl.ds(NSUB * SEG_PER_TILE, SEG_TAIL)])

    @pl.when(sid == 0)
    def _():
        pltpu.sync_copy(den_sh, den_out.at[cid])


def _sc_scatter(P, E2, idx, zeros):
    f = functools.partial(
        pl.kernel,
        mesh=plsc.VectorSubcoreMesh(core_axis_name="c", subcore_axis_name="s"),
        compiler_params=pltpu.CompilerParams(needs_layout_passes=False),
        out_type=[
            jax.ShapeDtypeStruct((NCORE, NUM_SEG, D_V), jnp.float32),
            jax.ShapeDtypeStruct((NCORE, DEN_ROWS, 128), jnp.float32),
        ],
        scratch_types=(
            [pltpu.VMEM((CH, D_V), jnp.float32) for _ in range(NBUF)]
            + [pltpu.VMEM((CH,), jnp.float32) for _ in range(NBUF)]
            + [pltpu.VMEM((CH,), jnp.int32) for _ in range(NBUF)]
            + [pltpu.VMEM((DEN_ROWS, 128), jnp.float32),
               pltpu.VMEM((DEN_ROWS,), jnp.int32),
               pltpu.VMEM_SHARED((NUM_SEG, D_V), jnp.float32),
               pltpu.VMEM_SHARED((DEN_ROWS, 128), jnp.float32)]
            + [pltpu.SemaphoreType.DMA for _ in range(2 * NBUF)]
        ),
    )(_sc_scatter_body)
    return f(P, E2, idx, zeros)


def _tc_combine_body(acc_ref, den_ref, out_ref):
    num = acc_ref[0] + acc_ref[1]
    den = jnp.sum(den_ref[...], axis=0)[:, None]
    out_ref[...] = jnp.where(den > 0.0, num / den, 0.0)


def _tc_combine(acc, den):
    return pl.pallas_call(
        _tc_combine_body,
        out_shape=jax.ShapeDtypeStruct((NUM_SEG, D_V), jnp.float32),
    )(acc, den)


def kernel(Q, K, V, W_w, U_w, V_w, batch_index):
    vw8 = jnp.broadcast_to(V_w.reshape(1, HID), (8, HID))
    idx = batch_index.astype(jnp.int32)
    zeros = jnp.zeros((ZR, D_V), jnp.float32)
    P, E2 = _tc_dense(Q, K, V, W_w, U_w, vw8)
    acc, den = _sc_scatter(P, E2.reshape(N), idx, zeros)
    den2 = den.reshape(NCORE, DEN_ROWS * 128)[:, :NUM_SEG]
    return _tc_combine(acc, den2)


# speedup vs baseline: 7.8780x; 1.2223x over previous
import functools

import jax
import jax.numpy as jnp
from jax import lax
from jax.experimental import pallas as pl
from jax.experimental.pallas import tpu as pltpu
from jax.experimental.pallas import tpu_sc as plsc

N = 320000
NUM_SEG = 10000
Q_SIZE = 128
K_SIZE = 128
HID = 64
D_V = 128


BLK = 2560
GRID1 = N // BLK
EROWS = BLK // 128


def _tc_dense_body(q_ref, k_ref, v_ref, w_ref, u_ref, vw_ref, p_ref, e_ref):
    qw = jnp.dot(q_ref[...], w_ref[...], preferred_element_type=jnp.float32)
    ku = jnp.dot(k_ref[...], u_ref[...], preferred_element_type=jnp.float32)
    t = jnp.tanh(qw + ku)
    beta = jnp.sum(t * vw_ref[0:1, :], axis=1, keepdims=True)
    e = jnp.exp(beta)
    p_ref[...] = v_ref[...] * e
    e_ref[...] = jnp.reshape(e[:, 0], (1, EROWS, 128))


def _tc_dense(Q, K, V, W_w, U_w, vw8):
    return pl.pallas_call(
        _tc_dense_body,
        grid=(GRID1,),
        in_specs=[
            pl.BlockSpec((BLK, Q_SIZE), lambda i: (i, 0)),
            pl.BlockSpec((BLK, K_SIZE), lambda i: (i, 0)),
            pl.BlockSpec((BLK, D_V), lambda i: (i, 0)),
            pl.BlockSpec((Q_SIZE, HID), lambda i: (0, 0)),
            pl.BlockSpec((K_SIZE, HID), lambda i: (0, 0)),
            pl.BlockSpec((8, HID), lambda i: (0, 0)),
        ],
        out_specs=[
            pl.BlockSpec((BLK, D_V), lambda i: (i, 0)),
            pl.BlockSpec((1, EROWS, 128), lambda i: (i, 0, 0)),
        ],
        out_shape=[
            jax.ShapeDtypeStruct((N, D_V), jnp.float32),
            jax.ShapeDtypeStruct((GRID1, EROWS, 128), jnp.float32),
        ],
    )(Q, K, V, W_w, U_w, vw8)


NCORE = 2
NSUB = 16
NW = NCORE * NSUB
CH = 80
ROWS_PER_TILE = N // NW
ITERS = ROWS_PER_TILE // CH
NBUF = 3
PREF = 2
DEN_ROWS = 80
SEG_PER_TILE = 624
SEG_TAIL = NUM_SEG - NSUB * SEG_PER_TILE
ZR = 104


def _sc_scatter_body(p_hbm, e_hbm, idx_hbm, zeros_hbm,
                     acc_out, den_out, *rest):
    data = list(rest[0:NBUF])
    ev = list(rest[NBUF:2 * NBUF])
    xv = list(rest[2 * NBUF:3 * NBUF])
    den_v = rest[3 * NBUF]
    idx_id = rest[3 * NBUF + 1]
    acc_sh = rest[3 * NBUF + 2]
    den_sh = rest[3 * NBUF + 3]
    isem = list(rest[3 * NBUF + 4:4 * NBUF + 4])
    ssem = list(rest[4 * NBUF + 4:5 * NBUF + 4])

    cid = lax.axis_index("c")
    sid = lax.axis_index("s")
    wid = cid * NSUB + sid
    row0 = wid * ROWS_PER_TILE

    def in_copies(i, b):
        base = row0 + i * CH
        return (
            pltpu.make_async_copy(p_hbm.at[pl.ds(base, CH)], data[b], isem[b]),
            pltpu.make_async_copy(e_hbm.at[pl.ds(base, CH)], ev[b], isem[b]),
            pltpu.make_async_copy(idx_hbm.at[pl.ds(base, CH)], xv[b], isem[b]),
        )

    def scat_copy(i, b):
        del i
        return pltpu.make_async_copy(data[b], acc_sh.at[xv[b]], ssem[b])

    for z in range(SEG_PER_TILE // ZR):
        pltpu.sync_copy(zeros_hbm,
                        acc_sh.at[pl.ds(sid * SEG_PER_TILE + z * ZR, ZR)])

    @pl.when(sid == NSUB - 1)
    def _():
        pltpu.sync_copy(zeros_hbm.at[pl.ds(0, SEG_TAIL)],
                        acc_sh.at[pl.ds(NSUB * SEG_PER_TILE, SEG_TAIL)])

    pltpu.sync_copy(zeros_hbm.at[pl.ds(0, DEN_ROWS)], den_v)
    for j in range(DEN_ROWS // 16):
        idx_id[pl.ds(j * 16, 16)] = lax.iota(jnp.int32, 16) + (j * 16)

    @pl.when(sid == 0)
    def _():
        pltpu.sync_copy(zeros_hbm.at[pl.ds(0, DEN_ROWS)], den_sh)

    plsc.subcore_barrier()

    for b in range(PREF):
        for c in in_copies(b, b):
            c.start()

    def _step(i, b):
        for c in in_copies(i, b):
            c.wait()
        for j in range(CH // 16):
            sl = pl.ds(j * 16, 16)
            idx16 = xv[b][sl]
            hi16 = lax.shift_right_logical(idx16, 7)
            lo16 = lax.bitwise_and(idx16, 127)
            plsc.addupdate_scatter(den_v, [hi16, lo16], ev[b][sl])
        scat_copy(i, b).start(add=True)

    LAG = NBUF - PREF

    def _round(o, carry):
        for b in range(NBUF):
            i = o * NBUF + b
            _step(i, b)
            bp = (b + PREF) % NBUF

            @pl.when(i >= LAG)
            def _():
                scat_copy(i - LAG, bp).wait()

            @pl.when(i + PREF < ITERS)
            def _():
                for c in in_copies(i + PREF, bp):
                    c.start()
        return carry

    MAIN = (ITERS // NBUF) * NBUF
    lax.fori_loop(0, ITERS // NBUF, _round, 0)

    for i in range(MAIN, ITERS):
        _step(i, i % NBUF)
        scat_copy(i - LAG, (i + PREF) % NBUF).wait()

    for i in range(ITERS - LAG, ITERS):
        scat_copy(i, i % NBUF).wait()

    pltpu.sync_copy(den_v, den_sh.at[idx_id], add=True)

    plsc.subcore_barrier()

    row0 = sid * SEG_PER_TILE
    pltpu.sync_copy(acc_sh.at[pl.ds(row0, SEG_PER_TILE)],
                    acc_out.at[cid, pl.ds(row0, SEG_PER_TILE)])

    @pl.when(sid == NSUB - 1)
    def _():
        pltpu.sync_copy(acc_sh.at[pl.ds(NSUB * SEG_PER_TILE, SEG_TAIL)],
                        acc_out.at[cid, pl.ds(NSUB * SEG_PER_TILE, SEG_TAIL)])

    @pl.when(sid == 0)
    def _():
        pltpu.sync_copy(den_sh, den_out.at[cid])


def _sc_scatter(P, E2, idx, zeros):
    f = functools.partial(
        pl.kernel,
        mesh=plsc.VectorSubcoreMesh(core_axis_name="c", subcore_axis_name="s"),
        compiler_params=pltpu.CompilerParams(needs_layout_passes=False),
        out_type=[
            jax.ShapeDtypeStruct((NCORE, NUM_SEG, D_V), jnp.float32),
            jax.ShapeDtypeStruct((NCORE, DEN_ROWS, 128), jnp.float32),
        ],
        scratch_types=(
            [pltpu.VMEM((CH, D_V), jnp.float32) for _ in range(NBUF)]
            + [pltpu.VMEM((CH,), jnp.float32) for _ in range(NBUF)]
            + [pltpu.VMEM((CH,), jnp.int32) for _ in range(NBUF)]
            + [pltpu.VMEM((DEN_ROWS, 128), jnp.float32),
               pltpu.VMEM((DEN_ROWS,), jnp.int32),
               pltpu.VMEM_SHARED((NUM_SEG, D_V), jnp.float32),
               pltpu.VMEM_SHARED((DEN_ROWS, 128), jnp.float32)]
            + [pltpu.SemaphoreType.DMA for _ in range(2 * NBUF)]
        ),
    )(_sc_scatter_body)
    return f(P, E2, idx, zeros)


def _tc_combine_body(acc_ref, den_ref, out_ref):
    num = acc_ref[0] + acc_ref[1]
    den = jnp.sum(den_ref[...], axis=0)[:, None]
    out_ref[...] = jnp.where(den > 0.0, num / den, 0.0)


def _tc_combine(acc, den):
    return pl.pallas_call(
        _tc_combine_body,
        out_shape=jax.ShapeDtypeStruct((NUM_SEG, D_V), jnp.float32),
    )(acc, den)


def kernel(Q, K, V, W_w, U_w, V_w, batch_index):
    vw8 = jnp.broadcast_to(V_w.reshape(1, HID), (8, HID))
    idx = batch_index.astype(jnp.int32)
    zeros = jnp.zeros((ZR, D_V), jnp.float32)
    P, E2 = _tc_dense(Q, K, V, W_w, U_w, vw8)
    acc, den = _sc_scatter(P, E2.reshape(N), idx, zeros)
    den2 = den.reshape(NCORE, DEN_ROWS * 128)[:, :NUM_SEG]
    return _tc_combine(acc, den2)


# speedup vs baseline: 8.4709x; 1.0753x over previous
import functools

import jax
import jax.numpy as jnp
from jax import lax
from jax.experimental import pallas as pl
from jax.experimental.pallas import tpu as pltpu
from jax.experimental.pallas import tpu_sc as plsc

N = 320000
NUM_SEG = 10000
Q_SIZE = 128
K_SIZE = 128
HID = 64
D_V = 128


BLK = 6400
GRID1 = N // BLK
EROWS = BLK // 128


def _tc_dense_body(q_ref, k_ref, v_ref, w_ref, u_ref, vw_ref, p_ref, e_ref):
    qw = jnp.dot(q_ref[...], w_ref[...], preferred_element_type=jnp.float32)
    ku = jnp.dot(k_ref[...], u_ref[...], preferred_element_type=jnp.float32)
    t = jnp.tanh(qw + ku)
    beta = jnp.sum(t * vw_ref[0:1, :], axis=1, keepdims=True)
    e = jnp.exp(beta)
    p_ref[...] = v_ref[...] * e
    e_ref[...] = jnp.reshape(e[:, 0], (1, EROWS, 128))


def _tc_dense(Q, K, V, W_w, U_w, vw8):
    return pl.pallas_call(
        _tc_dense_body,
        grid=(GRID1,),
        in_specs=[
            pl.BlockSpec((BLK, Q_SIZE), lambda i: (i, 0)),
            pl.BlockSpec((BLK, K_SIZE), lambda i: (i, 0)),
            pl.BlockSpec((BLK, D_V), lambda i: (i, 0)),
            pl.BlockSpec((Q_SIZE, HID), lambda i: (0, 0)),
            pl.BlockSpec((K_SIZE, HID), lambda i: (0, 0)),
            pl.BlockSpec((8, HID), lambda i: (0, 0)),
        ],
        out_specs=[
            pl.BlockSpec((BLK, D_V), lambda i: (i, 0)),
            pl.BlockSpec((1, EROWS, 128), lambda i: (i, 0, 0)),
        ],
        out_shape=[
            jax.ShapeDtypeStruct((N, D_V), jnp.float32),
            jax.ShapeDtypeStruct((GRID1, EROWS, 128), jnp.float32),
        ],
    )(Q, K, V, W_w, U_w, vw8)


NCORE = 2
NSUB = 16
NW = NCORE * NSUB
CH = 80
ROWS_PER_TILE = N // NW
ITERS = ROWS_PER_TILE // CH
NBUF = 3
PREF = 2
DEN_ROWS = 80
SEG_PER_TILE = 624
SEG_TAIL = NUM_SEG - NSUB * SEG_PER_TILE
ZR = 104


def _sc_scatter_body(p_hbm, e_hbm, idx_hbm, zeros_hbm,
                     acc_out, den_out, *rest):
    data = list(rest[0:NBUF])
    ev = list(rest[NBUF:2 * NBUF])
    xv = list(rest[2 * NBUF:3 * NBUF])
    den_v = rest[3 * NBUF]
    idx_id = rest[3 * NBUF + 1]
    acc_sh = rest[3 * NBUF + 2]
    den_sh = rest[3 * NBUF + 3]
    isem = list(rest[3 * NBUF + 4:4 * NBUF + 4])
    ssem = list(rest[4 * NBUF + 4:5 * NBUF + 4])

    cid = lax.axis_index("c")
    sid = lax.axis_index("s")
    wid = cid * NSUB + sid
    row0 = wid * ROWS_PER_TILE

    def in_copies(i, b):
        base = row0 + i * CH
        return (
            pltpu.make_async_copy(p_hbm.at[pl.ds(base, CH)], data[b], isem[b]),
            pltpu.make_async_copy(e_hbm.at[pl.ds(base, CH)], ev[b], isem[b]),
            pltpu.make_async_copy(idx_hbm.at[pl.ds(base, CH)], xv[b], isem[b]),
        )

    def scat_copy(i, b):
        del i
        return pltpu.make_async_copy(data[b], acc_sh.at[xv[b]], ssem[b])

    for z in range(SEG_PER_TILE // ZR):
        pltpu.sync_copy(zeros_hbm,
                        acc_sh.at[pl.ds(sid * SEG_PER_TILE + z * ZR, ZR)])

    @pl.when(sid == NSUB - 1)
    def _():
        pltpu.sync_copy(zeros_hbm.at[pl.ds(0, SEG_TAIL)],
                        acc_sh.at[pl.ds(NSUB * SEG_PER_TILE, SEG_TAIL)])

    pltpu.sync_copy(zeros_hbm.at[pl.ds(0, DEN_ROWS)], den_v)
    for j in range(DEN_ROWS // 16):
        idx_id[pl.ds(j * 16, 16)] = lax.iota(jnp.int32, 16) + (j * 16)

    @pl.when(sid == 0)
    def _():
        pltpu.sync_copy(zeros_hbm.at[pl.ds(0, DEN_ROWS)], den_sh)

    plsc.subcore_barrier()

    for b in range(PREF):
        for c in in_copies(b, b):
            c.start()

    def _step(i, b):
        for c in in_copies(i, b):
            c.wait()
        for j in range(CH // 16):
            sl = pl.ds(j * 16, 16)
            idx16 = xv[b][sl]
            hi16 = lax.shift_right_logical(idx16, 7)
            lo16 = lax.bitwise_and(idx16, 127)
            plsc.addupdate_scatter(den_v, [hi16, lo16], ev[b][sl])
        scat_copy(i, b).start(add=True)

    LAG = NBUF - PREF

    def _round(o, carry):
        for b in range(NBUF):
            i = o * NBUF + b
            _step(i, b)
            bp = (b + PREF) % NBUF

            @pl.when(i >= LAG)
            def _():
                scat_copy(i - LAG, bp).wait()

            @pl.when(i + PREF < ITERS)
            def _():
                for c in in_copies(i + PREF, bp):
                    c.start()
        return carry

    MAIN = (ITERS // NBUF) * NBUF
    lax.fori_loop(0, ITERS // NBUF, _round, 0)

    for i in range(MAIN, ITERS):
        _step(i, i % NBUF)
        scat_copy(i - LAG, (i + PREF) % NBUF).wait()

    for i in range(ITERS - LAG, ITERS):
        scat_copy(i, i % NBUF).wait()

    pltpu.sync_copy(den_v, den_sh.at[idx_id], add=True)

    plsc.subcore_barrier()

    row0 = sid * SEG_PER_TILE
    pltpu.sync_copy(acc_sh.at[pl.ds(row0, SEG_PER_TILE)],
                    acc_out.at[cid, pl.ds(row0, SEG_PER_TILE)])

    @pl.when(sid == NSUB - 1)
    def _():
        pltpu.sync_copy(acc_sh.at[pl.ds(NSUB * SEG_PER_TILE, SEG_TAIL)],
                        acc_out.at[cid, pl.ds(NSUB * SEG_PER_TILE, SEG_TAIL)])

    @pl.when(sid == 0)
    def _():
        pltpu.sync_copy(den_sh, den_out.at[cid])


def _sc_scatter(P, E2, idx, zeros):
    f = functools.partial(
        pl.kernel,
        mesh=plsc.VectorSubcoreMesh(core_axis_name="c", subcore_axis_name="s"),
        compiler_params=pltpu.CompilerParams(needs_layout_passes=False),
        out_type=[
            jax.ShapeDtypeStruct((NCORE, NUM_SEG, D_V), jnp.float32),
            jax.ShapeDtypeStruct((NCORE, DEN_ROWS, 128), jnp.float32),
        ],
        scratch_types=(
            [pltpu.VMEM((CH, D_V), jnp.float32) for _ in range(NBUF)]
            + [pltpu.VMEM((CH,), jnp.float32) for _ in range(NBUF)]
            + [pltpu.VMEM((CH,), jnp.int32) for _ in range(NBUF)]
            + [pltpu.VMEM((DEN_ROWS, 128), jnp.float32),
               pltpu.VMEM((DEN_ROWS,), jnp.int32),
               pltpu.VMEM_SHARED((NUM_SEG, D_V), jnp.float32),
               pltpu.VMEM_SHARED((DEN_ROWS, 128), jnp.float32)]
            + [pltpu.SemaphoreType.DMA for _ in range(2 * NBUF)]
        ),
    )(_sc_scatter_body)
    return f(P, E2, idx, zeros)


def _tc_combine_body(acc_ref, den_ref, out_ref):
    num = acc_ref[0] + acc_ref[1]
    den = jnp.sum(den_ref[...], axis=0)[:, None]
    out_ref[...] = jnp.where(den > 0.0, num / den, 0.0)


def _tc_combine(acc, den):
    return pl.pallas_call(
        _tc_combine_body,
        out_shape=jax.ShapeDtypeStruct((NUM_SEG, D_V), jnp.float32),
    )(acc, den)


def kernel(Q, K, V, W_w, U_w, V_w, batch_index):
    vw8 = jnp.broadcast_to(V_w.reshape(1, HID), (8, HID))
    idx = batch_index.astype(jnp.int32)
    zeros = jnp.zeros((ZR, D_V), jnp.float32)
    P, E2 = _tc_dense(Q, K, V, W_w, U_w, vw8)
    acc, den = _sc_scatter(P, E2.reshape(N), idx, zeros)
    den2 = den.reshape(NCORE, DEN_ROWS * 128)[:, :NUM_SEG]
    return _tc_combine(acc, den2)


# speedup vs baseline: 8.4824x; 1.0014x over previous
import functools

import jax
import jax.numpy as jnp
from jax import lax
from jax.experimental import pallas as pl
from jax.experimental.pallas import tpu as pltpu
from jax.experimental.pallas import tpu_sc as plsc

N = 320000
NUM_SEG = 10000
Q_SIZE = 128
K_SIZE = 128
HID = 64
D_V = 128


BLK = 12800
GRID1 = N // BLK
EROWS = BLK // 128


def _tc_dense_body(q_ref, k_ref, v_ref, w_ref, u_ref, vw_ref, p_ref, e_ref):
    qw = jnp.dot(q_ref[...], w_ref[...], preferred_element_type=jnp.float32)
    ku = jnp.dot(k_ref[...], u_ref[...], preferred_element_type=jnp.float32)
    t = jnp.tanh(qw + ku)
    beta = jnp.sum(t * vw_ref[0:1, :], axis=1, keepdims=True)
    e = jnp.exp(beta)
    p_ref[...] = v_ref[...] * e
    e_ref[...] = jnp.reshape(e[:, 0], (1, EROWS, 128))


def _tc_dense(Q, K, V, W_w, U_w, vw8):
    return pl.pallas_call(
        _tc_dense_body,
        grid=(GRID1,),
        in_specs=[
            pl.BlockSpec((BLK, Q_SIZE), lambda i: (i, 0)),
            pl.BlockSpec((BLK, K_SIZE), lambda i: (i, 0)),
            pl.BlockSpec((BLK, D_V), lambda i: (i, 0)),
            pl.BlockSpec((Q_SIZE, HID), lambda i: (0, 0)),
            pl.BlockSpec((K_SIZE, HID), lambda i: (0, 0)),
            pl.BlockSpec((8, HID), lambda i: (0, 0)),
        ],
        out_specs=[
            pl.BlockSpec((BLK, D_V), lambda i: (i, 0)),
            pl.BlockSpec((1, EROWS, 128), lambda i: (i, 0, 0)),
        ],
        out_shape=[
            jax.ShapeDtypeStruct((N, D_V), jnp.float32),
            jax.ShapeDtypeStruct((GRID1, EROWS, 128), jnp.float32),
        ],
    )(Q, K, V, W_w, U_w, vw8)


NCORE = 2
NSUB = 16
NW = NCORE * NSUB
CH = 80
ROWS_PER_TILE = N // NW
ITERS = ROWS_PER_TILE // CH
NBUF = 3
PREF = 2
DEN_ROWS = 80
SEG_PER_TILE = 624
SEG_TAIL = NUM_SEG - NSUB * SEG_PER_TILE
ZR = 104


def _sc_scatter_body(p_hbm, e_hbm, idx_hbm, zeros_hbm,
                     acc_out, den_out, *rest):
    data = list(rest[0:NBUF])
    ev = list(rest[NBUF:2 * NBUF])
    xv = list(rest[2 * NBUF:3 * NBUF])
    den_v = rest[3 * NBUF]
    idx_id = rest[3 * NBUF + 1]
    acc_sh = rest[3 * NBUF + 2]
    den_sh = rest[3 * NBUF + 3]
    isem = list(rest[3 * NBUF + 4:4 * NBUF + 4])
    ssem = list(rest[4 * NBUF + 4:5 * NBUF + 4])

    cid = lax.axis_index("c")
    sid = lax.axis_index("s")
    wid = cid * NSUB + sid
    row0 = wid * ROWS_PER_TILE

    def in_copies(i, b):
        base = row0 + i * CH
        return (
            pltpu.make_async_copy(p_hbm.at[pl.ds(base, CH)], data[b], isem[b]),
            pltpu.make_async_copy(e_hbm.at[pl.ds(base, CH)], ev[b], isem[b]),
            pltpu.make_async_copy(idx_hbm.at[pl.ds(base, CH)], xv[b], isem[b]),
        )

    def scat_copy(i, b):
        del i
        return pltpu.make_async_copy(data[b], acc_sh.at[xv[b]], ssem[b])

    for z in range(SEG_PER_TILE // ZR):
        pltpu.sync_copy(zeros_hbm,
                        acc_sh.at[pl.ds(sid * SEG_PER_TILE + z * ZR, ZR)])

    @pl.when(sid == NSUB - 1)
    def _():
        pltpu.sync_copy(zeros_hbm.at[pl.ds(0, SEG_TAIL)],
                        acc_sh.at[pl.ds(NSUB * SEG_PER_TILE, SEG_TAIL)])

    pltpu.sync_copy(zeros_hbm.at[pl.ds(0, DEN_ROWS)], den_v)
    for j in range(DEN_ROWS // 16):
        idx_id[pl.ds(j * 16, 16)] = lax.iota(jnp.int32, 16) + (j * 16)

    @pl.when(sid == 0)
    def _():
        pltpu.sync_copy(zeros_hbm.at[pl.ds(0, DEN_ROWS)], den_sh)

    plsc.subcore_barrier()

    for b in range(PREF):
        for c in in_copies(b, b):
            c.start()

    def _step(i, b):
        for c in in_copies(i, b):
            c.wait()
        for j in range(CH // 16):
            sl = pl.ds(j * 16, 16)
            idx16 = xv[b][sl]
            hi16 = lax.shift_right_logical(idx16, 7)
            lo16 = lax.bitwise_and(idx16, 127)
            plsc.addupdate_scatter(den_v, [hi16, lo16], ev[b][sl])
        scat_copy(i, b).start(add=True)

    LAG = NBUF - PREF

    def _round(o, carry):
        for b in range(NBUF):
            i = o * NBUF + b
            _step(i, b)
            bp = (b + PREF) % NBUF

            @pl.when(i >= LAG)
            def _():
                scat_copy(i - LAG, bp).wait()

            @pl.when(i + PREF < ITERS)
            def _():
                for c in in_copies(i + PREF, bp):
                    c.start()
        return carry

    MAIN = (ITERS // NBUF) * NBUF
    lax.fori_loop(0, ITERS // NBUF, _round, 0)

    for i in range(MAIN, ITERS):
        _step(i, i % NBUF)
        scat_copy(i - LAG, (i + PREF) % NBUF).wait()

    for i in range(ITERS - LAG, ITERS):
        scat_copy(i, i % NBUF).wait()

    pltpu.sync_copy(den_v, den_sh.at[idx_id], add=True)

    plsc.subcore_barrier()

    row0 = sid * SEG_PER_TILE
    pltpu.sync_copy(acc_sh.at[pl.ds(row0, SEG_PER_TILE)],
                    acc_out.at[cid, pl.ds(row0, SEG_PER_TILE)])

    @pl.when(sid == NSUB - 1)
    def _():
        pltpu.sync_copy(acc_sh.at[pl.ds(NSUB * SEG_PER_TILE, SEG_TAIL)],
                        acc_out.at[cid, pl.ds(NSUB * SEG_PER_TILE, SEG_TAIL)])

    @pl.when(sid == 0)
    def _():
        pltpu.sync_copy(den_sh, den_out.at[cid])


def _sc_scatter(P, E2, idx, zeros):
    f = functools.partial(
        pl.kernel,
        mesh=plsc.VectorSubcoreMesh(core_axis_name="c", subcore_axis_name="s"),
        compiler_params=pltpu.CompilerParams(needs_layout_passes=False),
        out_type=[
            jax.ShapeDtypeStruct((NCORE, NUM_SEG, D_V), jnp.float32),
            jax.ShapeDtypeStruct((NCORE, DEN_ROWS, 128), jnp.float32),
        ],
        scratch_types=(
            [pltpu.VMEM((CH, D_V), jnp.float32) for _ in range(NBUF)]
            + [pltpu.VMEM((CH,), jnp.float32) for _ in range(NBUF)]
            + [pltpu.VMEM((CH,), jnp.int32) for _ in range(NBUF)]
            + [pltpu.VMEM((DEN_ROWS, 128), jnp.float32),
               pltpu.VMEM((DEN_ROWS,), jnp.int32),
               pltpu.VMEM_SHARED((NUM_SEG, D_V), jnp.float32),
               pltpu.VMEM_SHARED((DEN_ROWS, 128), jnp.float32)]
            + [pltpu.SemaphoreType.DMA for _ in range(2 * NBUF)]
        ),
    )(_sc_scatter_body)
    return f(P, E2, idx, zeros)


def _tc_combine_body(acc_ref, den_ref, out_ref):
    num = acc_ref[0] + acc_ref[1]
    den = jnp.sum(den_ref[...], axis=0)[:, None]
    out_ref[...] = jnp.where(den > 0.0, num / den, 0.0)


def _tc_combine(acc, den):
    return pl.pallas_call(
        _tc_combine_body,
        out_shape=jax.ShapeDtypeStruct((NUM_SEG, D_V), jnp.float32),
    )(acc, den)


def kernel(Q, K, V, W_w, U_w, V_w, batch_index):
    vw8 = jnp.broadcast_to(V_w.reshape(1, HID), (8, HID))
    idx = batch_index.astype(jnp.int32)
    zeros = jnp.zeros((ZR, D_V), jnp.float32)
    P, E2 = _tc_dense(Q, K, V, W_w, U_w, vw8)
    acc, den = _sc_scatter(P, E2.reshape(N), idx, zeros)
    den2 = den.reshape(NCORE, DEN_ROWS * 128)[:, :NUM_SEG]
    return _tc_combine(acc, den2)
